# baseline (speedup 1.0000x reference)
import functools

import jax
import jax.numpy as jnp
from jax.experimental import pallas as pl
from jax.experimental.pallas import tpu as pltpu

_NCHUNK = 8


def _body(cap, size, s_ref, mem_ref, val_ref, out_ref, csem, vsem):
    ch = cap // _NCHUNK
    bulk = [
        pltpu.make_async_copy(
            mem_ref.at[pl.ds(c * ch, ch), :],
            out_ref.at[pl.ds(c * ch, ch), :],
            csem,
        )
        for c in range(_NCHUNK)
    ]
    for d in bulk:
        d.start()
    for d in bulk:
        d.wait()

    s0 = s_ref[0]
    n1 = jnp.minimum(jnp.int32(size), cap - s0)
    nbits = size.bit_length()

    def segment(length, src_base, dst_base, sem):
        copies = []
        off = jnp.int32(0)
        for k in reversed(range(nbits)):
            ln = 1 << k
            bit = (length & ln) != 0
            d = pltpu.make_async_copy(
                val_ref.at[pl.ds(src_base + off, ln), :],
                out_ref.at[pl.ds(dst_base + off, ln), :],
                sem,
            )

            @pl.when(bit)
            def _start(d=d):
                d.start()

            copies.append((bit, d))
            off = off + jnp.where(bit, jnp.int32(ln), jnp.int32(0))
        return copies

    seg = segment(n1, jnp.int32(0), s0, vsem)
    seg += segment(jnp.int32(size) - n1, n1, jnp.int32(0), vsem)
    for bit, d in seg:

        @pl.when(bit)
        def _wait(d=d):
            d.wait()


def kernel(mem, val, store_index):
    cap, d = mem.shape
    size = min(val.shape[0], cap)
    assert cap % _NCHUNK == 0

    s0 = jnp.remainder(jnp.asarray(store_index, jnp.int32), cap).reshape(1)

    body = functools.partial(_body, cap, size)
    return pl.pallas_call(
        body,
        out_shape=jax.ShapeDtypeStruct((cap, d), mem.dtype),
        in_specs=[
            pl.BlockSpec(memory_space=pltpu.SMEM),
            pl.BlockSpec(memory_space=pl.ANY),
            pl.BlockSpec(memory_space=pl.ANY),
        ],
        out_specs=pl.BlockSpec(memory_space=pl.ANY),
        scratch_shapes=[pltpu.SemaphoreType.DMA, pltpu.SemaphoreType.DMA],
    )(s0, mem, val)


# speedup vs baseline: 17.4570x; 17.4570x over previous
import functools

import jax
import jax.numpy as jnp
from jax.experimental import pallas as pl
from jax.experimental.pallas import tpu as pltpu


def _body(cap, size, s_ref, mem_ref, val_ref, out_ref, vsem):
    del mem_ref
    s0 = s_ref[0]
    n1 = jnp.minimum(jnp.int32(size), cap - s0)
    nbits = size.bit_length()

    def segment(length, src_base, dst_base):
        copies = []
        off = jnp.int32(0)
        for k in reversed(range(nbits)):
            ln = 1 << k
            bit = (length & ln) != 0
            d = pltpu.make_async_copy(
                val_ref.at[pl.ds(src_base + off, ln), :],
                out_ref.at[pl.ds(dst_base + off, ln), :],
                vsem,
            )

            @pl.when(bit)
            def _start(d=d):
                d.start()

            copies.append((bit, d))
            off = off + jnp.where(bit, jnp.int32(ln), jnp.int32(0))
        return copies

    seg = segment(n1, jnp.int32(0), s0)
    seg += segment(jnp.int32(size) - n1, n1, jnp.int32(0))
    for bit, d in seg:

        @pl.when(bit)
        def _wait(d=d):
            d.wait()


def kernel(mem, val, store_index):
    cap, d = mem.shape
    size = min(val.shape[0], cap)

    s0 = jnp.remainder(jnp.asarray(store_index, jnp.int32), cap).reshape(1)

    body = functools.partial(_body, cap, size)
    return pl.pallas_call(
        body,
        out_shape=jax.ShapeDtypeStruct((cap, d), mem.dtype),
        in_specs=[
            pl.BlockSpec(memory_space=pltpu.SMEM),
            pl.BlockSpec(memory_space=pl.ANY),
            pl.BlockSpec(memory_space=pl.ANY),
        ],
        out_specs=pl.BlockSpec(memory_space=pl.ANY),
        input_output_aliases={1: 0},
        scratch_shapes=[pltpu.SemaphoreType.DMA],
    )(s0, mem, val)
